# Initial kernel scaffold; baseline (speedup 1.0000x reference)
import jax
import jax.numpy as jnp
from jax.experimental import pallas as pl


def kernel(features):
    raise NotImplementedError("write your pallas kernel here")


# speedup vs baseline: 16.8506x; 16.8506x over previous
import jax
import jax.numpy as jnp
from jax.experimental import pallas as pl

_TOPK = 512
_ROWS_PER_BLOCK = 16
_INT_MIN = -(2**31)


def _order_key(x):
    bits = jax.lax.bitcast_convert_type(x, jnp.int32)
    return jnp.where(bits < 0, jnp.bitwise_xor(~bits, jnp.int32(_INT_MIN)), bits)


def _tc_body(x_ref, o_ref):
    x = x_ref[...]
    key = _order_key(x)
    r = x.shape[0]
    t = jnp.full((r, 1), _INT_MIN, dtype=jnp.int32)
    for i in range(32):
        bit = 31 - i
        cand = t + (jnp.int32(1) << jnp.int32(bit))
        cnt = jnp.sum((key >= cand).astype(jnp.int32), axis=1, keepdims=True)
        t = jnp.where(cnt >= _TOPK, cand, t)
    thr = jnp.maximum(t, 0)
    o_ref[...] = jnp.where(key >= thr, x, 0.0)


def kernel(features):
    b, l, d = features.shape
    n = l * d
    flat = features.reshape(b, n)
    r = _ROWS_PER_BLOCK
    out = pl.pallas_call(
        _tc_body,
        grid=(b // r,),
        in_specs=[pl.BlockSpec((r, n), lambda i: (i, 0))],
        out_specs=pl.BlockSpec((r, n), lambda i: (i, 0)),
        out_shape=jax.ShapeDtypeStruct((b, n), jnp.float32),
    )(flat)
    return out.reshape(b, l, d)
